# baseline (speedup 1.0000x reference)
import functools

import jax
import jax.numpy as jnp
from jax import lax
from jax.experimental import pallas as pl
from jax.experimental.pallas import tpu as pltpu
from jax.experimental.pallas import tpu_sc as plsc

ETA = 0.8
GAMMA = 0.8

GRID_L = 1024
N_ROWS = GRID_L * GRID_L
PLANE = 2 * N_ROWS
NUM_CORES = 2
NUM_SUBCORES = 16
NW = NUM_CORES * NUM_SUBCORES
ROWS_PER_W = N_ROWS // NW
CHUNK = 8192
NCHUNK = ROWS_PER_W // CHUNK
GROUPS = CHUNK // 16

_MESH = plsc.VectorSubcoreMesh(
    core_axis_name="c", subcore_axis_name="s",
    num_cores=NUM_CORES, num_subcores=NUM_SUBCORES)

_SET = [
    pltpu.VMEM((CHUNK * 2,), jnp.float32),
    pltpu.VMEM((CHUNK * 2,), jnp.float32),
    pltpu.VMEM((CHUNK,), jnp.int32),
    pltpu.VMEM((CHUNK,), jnp.int32),
    pltpu.VMEM((CHUNK,), jnp.float32),
]


@functools.partial(
    pl.kernel,
    out_type=jax.ShapeDtypeStruct((N_ROWS * 4,), jnp.float32),
    mesh=_MESH,
    compiler_params=pltpu.CompilerParams(needs_layout_passes=False),
    scratch_types=_SET + _SET + [
        pltpu.SemaphoreType.DMA,
        pltpu.SemaphoreType.DMA,
        pltpu.SemaphoreType.DMA,
        pltpu.SemaphoreType.DMA,
        pltpu.SemaphoreType.DMA,
        pltpu.SemaphoreType.DMA,
        pltpu.SemaphoreType.DMA,
        pltpu.SemaphoreType.DMA,
        pltpu.SemaphoreType.DMA,
        pltpu.SemaphoreType.DMA,
    ],
)
def _spgg_update(a_hbm, b_hbm, p_hbm, q_hbm, out_hbm,
                 q0a, q1a, aa, ba, pa, q0b, q1b, ab, bb, pb,
                 sing0, sing1, sq00, sq01, sq10, sq11,
                 so00, so01, so10, so11):
    wid = lax.axis_index("s") * NUM_CORES + lax.axis_index("c")
    bufs = [(q0a, q1a, aa, ba, pa, sing0, (sq00, sq01), (so00, so01)),
            (q0b, q1b, ab, bb, pb, sing1, (sq10, sq11), (so10, so11))]
    HALF = CHUNK

    def start_in_grid(ci):
        _, _, a_v, b_v, p_v, sing, _, _ = bufs[ci % 2]
        cbase = wid * ROWS_PER_W + ci * CHUNK
        return [
            pltpu.async_copy(a_hbm.at[pl.ds(cbase, CHUNK)], a_v, sing),
            pltpu.async_copy(b_hbm.at[pl.ds(cbase, CHUNK)], b_v, sing),
            pltpu.async_copy(p_hbm.at[pl.ds(cbase, CHUNK)], p_v, sing),
        ]

    def start_in_q(ci, h):
        q0_v, q1_v, _, _, _, _, sq, _ = bufs[ci % 2]
        cbase = wid * ROWS_PER_W + ci * CHUNK
        off = 2 * cbase + h * HALF
        return [
            pltpu.async_copy(q_hbm.at[pl.ds(off, HALF)],
                             q0_v.at[pl.ds(h * HALF, HALF)], sq[h]),
            pltpu.async_copy(q_hbm.at[pl.ds(PLANE + off, HALF)],
                             q1_v.at[pl.ds(h * HALF, HALF)], sq[h]),
        ]

    def start_out(ci, h):
        q0_v, q1_v, _, _, _, _, _, so = bufs[ci % 2]
        cbase = wid * ROWS_PER_W + ci * CHUNK
        off = 2 * cbase + h * HALF
        return [
            pltpu.async_copy(q0_v.at[pl.ds(h * HALF, HALF)],
                             out_hbm.at[pl.ds(off, HALF)], so[h]),
            pltpu.async_copy(q1_v.at[pl.ds(h * HALF, HALF)],
                             out_hbm.at[pl.ds(PLANE + off, HALF)], so[h]),
        ]

    def compute(ci, h):
        q0_v, q1_v, a_v, b_v, p_v, _, _, _ = bufs[ci % 2]

        @plsc.parallel_loop(h * GROUPS // 16, (h + 1) * GROUPS // 16, 1)
        def block_body(blk):
            rsub = blk >> 3
            colblk = blk & 7
            gb = colblk * 1024 + rsub * 128
            qb = rsub * 2048 + colblk * 256
            for lblk in range(8):
                go = gb + lblk * 16
                qo = qb + lblk * 16
                a = a_v[pl.ds(go, 16)]
                b = b_v[pl.ds(go, 16)]
                p_f = p_v[pl.ds(go, 16)]
                q00 = q0_v[pl.ds(qo, 16)]
                q01 = q0_v[pl.ds(qo + 128, 16)]
                q10 = q1_v[pl.ds(qo, 16)]
                q11 = q1_v[pl.ds(qo + 128, 16)]
                a0 = a == 0
                b0 = b == 0
                m = jnp.where(b0, jnp.maximum(q00, q01), jnp.maximum(q10, q11))
                old = jnp.where(a0, jnp.where(b0, q00, q01),
                                jnp.where(b0, q10, q11))
                upd = (1.0 - ETA) * old + ETA * (p_f + GAMMA * m)
                na0 = jnp.logical_not(a0)
                nb0 = jnp.logical_not(b0)
                q0_v[pl.ds(qo, 16)] = jnp.where(a0 & b0, upd, q00)
                q0_v[pl.ds(qo + 128, 16)] = jnp.where(a0 & nb0, upd, q01)
                q1_v[pl.ds(qo, 16)] = jnp.where(na0 & b0, upd, q10)
                q1_v[pl.ds(qo + 128, 16)] = jnp.where(na0 & nb0, upd, q11)

    grid_flight, q_flight, out_flight = {}, {}, {}
    grid_flight[0] = start_in_grid(0)
    q_flight[(0, 0)] = start_in_q(0, 0)
    q_flight[(0, 1)] = start_in_q(0, 1)
    for ci in range(NCHUNK):
        if ci + 1 < NCHUNK:
            if ci - 1 >= 0:
                for hh in (0, 1):
                    for hd in out_flight.pop((ci - 1, hh)):
                        hd.wait()
            grid_flight[ci + 1] = start_in_grid(ci + 1)
            q_flight[(ci + 1, 0)] = start_in_q(ci + 1, 0)
            q_flight[(ci + 1, 1)] = start_in_q(ci + 1, 1)
        for hd in grid_flight.pop(ci):
            hd.wait()
        for hh in (0, 1):
            for hd in q_flight.pop((ci, hh)):
                hd.wait()
            compute(ci, hh)
            out_flight[(ci, hh)] = start_out(ci, hh)
    for key in sorted(out_flight):
        for hd in out_flight[key]:
            hd.wait()


def kernel(type_t_matrix, type_t1_matrix, Q_tensor, profit_matrix):
    a_raw = type_t_matrix.reshape(128, 8, 8, 128).transpose(0, 2, 1, 3).reshape(-1)
    b_raw = type_t1_matrix.reshape(128, 8, 8, 128).transpose(0, 2, 1, 3).reshape(-1)
    p_raw = profit_matrix.reshape(128, 8, 8, 128).transpose(0, 2, 1, 3).reshape(-1)
    q_raw = Q_tensor.reshape(8192, 128, 2, 2).transpose(2, 0, 3, 1).reshape(-1)
    out_raw = _spgg_update(a_raw, b_raw, p_raw, q_raw)
    return (out_raw.reshape(2, 8192, 2, 128).transpose(1, 3, 0, 2)
            .reshape(N_ROWS, 2, 2))


# speedup vs baseline: 1.1240x; 1.1240x over previous
import functools

import jax
import jax.numpy as jnp
from jax import lax
from jax.experimental import pallas as pl
from jax.experimental.pallas import tpu as pltpu
from jax.experimental.pallas import tpu_sc as plsc

ETA = 0.8
GAMMA = 0.8

GRID_L = 1024
N_ROWS = GRID_L * GRID_L
PLANE = 2 * N_ROWS
NUM_CORES = 2
NUM_SUBCORES = 16
NW = NUM_CORES * NUM_SUBCORES
ROWS_PER_W = N_ROWS // NW
CHUNK = 8192
NCHUNK = ROWS_PER_W // CHUNK
GROUPS = CHUNK // 16

_MESH = plsc.VectorSubcoreMesh(
    core_axis_name="c", subcore_axis_name="s",
    num_cores=NUM_CORES, num_subcores=NUM_SUBCORES)

_SET = [
    pltpu.VMEM((CHUNK * 2,), jnp.float32),
    pltpu.VMEM((CHUNK * 2,), jnp.float32),
    pltpu.VMEM((CHUNK,), jnp.int32),
    pltpu.VMEM((CHUNK,), jnp.int32),
    pltpu.VMEM((CHUNK,), jnp.float32),
]


@functools.partial(
    pl.kernel,
    out_type=jax.ShapeDtypeStruct((N_ROWS * 4,), jnp.float32),
    mesh=_MESH,
    compiler_params=pltpu.CompilerParams(needs_layout_passes=False),
    scratch_types=_SET + _SET + [
        pltpu.SemaphoreType.DMA,
        pltpu.SemaphoreType.DMA,
        pltpu.SemaphoreType.DMA,
        pltpu.SemaphoreType.DMA,
        pltpu.SemaphoreType.DMA,
        pltpu.SemaphoreType.DMA,
        pltpu.SemaphoreType.DMA,
        pltpu.SemaphoreType.DMA,
        pltpu.SemaphoreType.DMA,
        pltpu.SemaphoreType.DMA,
    ],
)
def _spgg_update(a_hbm, b_hbm, p_hbm, q_hbm, out_hbm,
                 q0a, q1a, aa, ba, pa, q0b, q1b, ab, bb, pb,
                 sing0, sing1, sq00, sq01, sq10, sq11,
                 so00, so01, so10, so11):
    wid = lax.axis_index("s") * NUM_CORES + lax.axis_index("c")
    bufs = [(q0a, q1a, aa, ba, pa, sing0, (sq00, sq01), (so00, so01)),
            (q0b, q1b, ab, bb, pb, sing1, (sq10, sq11), (so10, so11))]
    HALF = CHUNK

    def start_in_grid(ci):
        _, _, a_v, b_v, p_v, sing, _, _ = bufs[ci % 2]
        cbase = wid * ROWS_PER_W + ci * CHUNK
        return [
            pltpu.async_copy(a_hbm.at[pl.ds(cbase, CHUNK)], a_v, sing),
            pltpu.async_copy(b_hbm.at[pl.ds(cbase, CHUNK)], b_v, sing),
            pltpu.async_copy(p_hbm.at[pl.ds(cbase, CHUNK)], p_v, sing),
        ]

    def start_in_q(ci, h):
        q0_v, q1_v, _, _, _, _, sq, _ = bufs[ci % 2]
        cbase = wid * ROWS_PER_W + ci * CHUNK
        off = 2 * cbase + h * HALF
        return [
            pltpu.async_copy(q_hbm.at[pl.ds(off, HALF)],
                             q0_v.at[pl.ds(h * HALF, HALF)], sq[h]),
            pltpu.async_copy(q_hbm.at[pl.ds(PLANE + off, HALF)],
                             q1_v.at[pl.ds(h * HALF, HALF)], sq[h]),
        ]

    def start_out(ci, h):
        q0_v, q1_v, _, _, _, _, _, so = bufs[ci % 2]
        cbase = wid * ROWS_PER_W + ci * CHUNK
        off = 2 * cbase + h * HALF
        return [
            pltpu.async_copy(q0_v.at[pl.ds(h * HALF, HALF)],
                             out_hbm.at[pl.ds(off, HALF)], so[h]),
            pltpu.async_copy(q1_v.at[pl.ds(h * HALF, HALF)],
                             out_hbm.at[pl.ds(PLANE + off, HALF)], so[h]),
        ]

    def compute(ci, h):
        q0_v, q1_v, a_v, b_v, p_v, _, _, _ = bufs[ci % 2]

        @plsc.parallel_loop(h * GROUPS // 2, (h + 1) * GROUPS // 2, 1, unroll=2)
        def group_body(g):
            rsub = g >> 6
            colblk = (g >> 3) & 7
            lblk = g & 7
            go = colblk * 1024 + rsub * 128 + lblk * 16
            qo = rsub * 2048 + colblk * 256 + lblk * 16
            a = a_v[pl.ds(go, 16)]
            b = b_v[pl.ds(go, 16)]
            p_f = p_v[pl.ds(go, 16)]
            q00 = q0_v[pl.ds(qo, 16)]
            q01 = q0_v[pl.ds(qo + 128, 16)]
            q10 = q1_v[pl.ds(qo, 16)]
            q11 = q1_v[pl.ds(qo + 128, 16)]
            a0 = a == 0
            b0 = b == 0
            m = jnp.where(b0, jnp.maximum(q00, q01), jnp.maximum(q10, q11))
            old = jnp.where(a0, jnp.where(b0, q00, q01),
                            jnp.where(b0, q10, q11))
            upd = (1.0 - ETA) * old + ETA * (p_f + GAMMA * m)
            na0 = jnp.logical_not(a0)
            nb0 = jnp.logical_not(b0)
            q0_v[pl.ds(qo, 16)] = jnp.where(a0 & b0, upd, q00)
            q0_v[pl.ds(qo + 128, 16)] = jnp.where(a0 & nb0, upd, q01)
            q1_v[pl.ds(qo, 16)] = jnp.where(na0 & b0, upd, q10)
            q1_v[pl.ds(qo + 128, 16)] = jnp.where(na0 & nb0, upd, q11)

    grid_flight, q_flight, out_flight = {}, {}, {}
    grid_flight[0] = start_in_grid(0)
    q_flight[(0, 0)] = start_in_q(0, 0)
    q_flight[(0, 1)] = start_in_q(0, 1)
    for ci in range(NCHUNK):
        if ci + 1 < NCHUNK:
            if ci - 1 >= 0:
                for hh in (0, 1):
                    for hd in out_flight.pop((ci - 1, hh)):
                        hd.wait()
            grid_flight[ci + 1] = start_in_grid(ci + 1)
            q_flight[(ci + 1, 0)] = start_in_q(ci + 1, 0)
            q_flight[(ci + 1, 1)] = start_in_q(ci + 1, 1)
        for hd in grid_flight.pop(ci):
            hd.wait()
        for hh in (0, 1):
            for hd in q_flight.pop((ci, hh)):
                hd.wait()
            compute(ci, hh)
            out_flight[(ci, hh)] = start_out(ci, hh)
    for key in sorted(out_flight):
        for hd in out_flight[key]:
            hd.wait()


def kernel(type_t_matrix, type_t1_matrix, Q_tensor, profit_matrix):
    a_raw = type_t_matrix.reshape(128, 8, 8, 128).transpose(0, 2, 1, 3).reshape(-1)
    b_raw = type_t1_matrix.reshape(128, 8, 8, 128).transpose(0, 2, 1, 3).reshape(-1)
    p_raw = profit_matrix.reshape(128, 8, 8, 128).transpose(0, 2, 1, 3).reshape(-1)
    q_raw = Q_tensor.reshape(8192, 128, 2, 2).transpose(2, 0, 3, 1).reshape(-1)
    out_raw = _spgg_update(a_raw, b_raw, p_raw, q_raw)
    return (out_raw.reshape(2, 8192, 2, 128).transpose(1, 3, 0, 2)
            .reshape(N_ROWS, 2, 2))
